# Initial kernel scaffold; baseline (speedup 1.0000x reference)
import jax
import jax.numpy as jnp
from jax.experimental import pallas as pl


def kernel(x, edge_index, W_root1, W_neigh1, b1, W_root2, W_neigh2, b2, P1, pb1, P2, pb2, P3, pb3):
    raise NotImplementedError("write your pallas kernel here")


# speedup vs baseline: 3.3480x; 3.3480x over previous
import functools

import jax
import jax.numpy as jnp
from jax import lax
from jax.experimental import pallas as pl
from jax.experimental.pallas import tpu as pltpu
from jax.experimental.pallas import tpu_sc as plsc

NN = 10000
EE = 320000
D = 128
PP = 1024
NC, NS = 2, 16
NW = NC * NS
CH = 128
NCHUNK = 80
IBLK = 8
E_PAD = NW * NCHUNK * CH
N_ACC = 10240
RPT = N_ACC // NS

_MESH = plsc.VectorSubcoreMesh(core_axis_name="c", subcore_axis_name="s")


def _make_seg_sum(with_cnt):
    out_type = [jax.ShapeDtypeStruct((NC, N_ACC, D), jnp.float32)]
    if with_cnt:
        out_type.append(jax.ShapeDtypeStruct((NC, N_ACC, D), jnp.float32))

    @functools.partial(
        pl.kernel,
        out_type=out_type,
        mesh=_MESH,
        scratch_types=[
            pltpu.VMEM((IBLK, CH), jnp.int32),
            pltpu.VMEM((IBLK, CH), jnp.int32),
            pltpu.VMEM((CH, D), jnp.float32),
            pltpu.VMEM((CH, D), jnp.float32),
            pltpu.VMEM_SHARED((N_ACC, D), jnp.float32),
            pltpu.SemaphoreType.DMA,
        ],
    )
    def seg(table_hbm, src_hbm, dst_hbm, zrows_hbm, ones_hbm,
            *out_and_scratch):
        if with_cnt:
            out_sum, out_cnt = out_and_scratch[:2]
            src_v, dst_v, rows_v, ones_v, acc_s, sem = out_and_scratch[2:]
        else:
            out_sum = out_and_scratch[0]
            src_v, dst_v, rows_v, ones_v, acc_s, sem = out_and_scratch[1:]
        c = lax.axis_index("c")
        s = lax.axis_index("s")
        w = s * NC + c
        base = s * RPT
        pltpu.sync_copy(zrows_hbm, acc_s.at[pl.ds(base, RPT)])
        plsc.subcore_barrier()

        def outer(o, carry):
            pltpu.sync_copy(src_hbm.at[pl.ds(w * NCHUNK + o * IBLK, IBLK)],
                            src_v)
            pltpu.sync_copy(dst_hbm.at[pl.ds(w * NCHUNK + o * IBLK, IBLK)],
                            dst_v)

            def body(j, carry2):
                pltpu.async_copy(table_hbm.at[src_v.at[j]], rows_v, sem).wait()
                pltpu.sync_copy(rows_v, acc_s.at[dst_v.at[j]], add=True)
                return carry2

            return lax.fori_loop(0, IBLK, body, carry)

        lax.fori_loop(0, NCHUNK // IBLK, outer, 0)
        plsc.subcore_barrier()
        pltpu.sync_copy(acc_s.at[pl.ds(base, RPT)],
                        out_sum.at[c, pl.ds(base, RPT)])

        if with_cnt:
            pltpu.sync_copy(ones_hbm, ones_v)
            plsc.subcore_barrier()
            pltpu.sync_copy(zrows_hbm, acc_s.at[pl.ds(base, RPT)])
            plsc.subcore_barrier()

            def outer_cnt(o, carry):
                pltpu.sync_copy(
                    dst_hbm.at[pl.ds(w * NCHUNK + o * IBLK, IBLK)], dst_v)

                def body_cnt(j, carry2):
                    pltpu.sync_copy(ones_v, acc_s.at[dst_v.at[j]], add=True)
                    return carry2

                return lax.fori_loop(0, IBLK, body_cnt, carry)

            lax.fori_loop(0, NCHUNK // IBLK, outer_cnt, 0)
            plsc.subcore_barrier()
            pltpu.sync_copy(acc_s.at[pl.ds(base, RPT)],
                            out_cnt.at[c, pl.ds(base, RPT)])

    return seg


_seg_sum_cnt = _make_seg_sum(True)
_seg_sum = _make_seg_sum(False)


PRED_ROWS = 2 * PP
PR_PER_W = PRED_ROWS // NW


@functools.partial(
    pl.kernel,
    out_type=jax.ShapeDtypeStruct((PRED_ROWS, D), jnp.float32),
    mesh=_MESH,
    scratch_types=[
        pltpu.VMEM((PR_PER_W,), jnp.int32),
        pltpu.VMEM((PR_PER_W, D), jnp.float32),
        pltpu.SemaphoreType.DMA,
    ],
)
def _gather_rows(h_hbm, idx_hbm, out, idx_v, rows_v, sem):
    c = lax.axis_index("c")
    s = lax.axis_index("s")
    base = (s * NC + c) * PR_PER_W
    pltpu.sync_copy(idx_hbm.at[pl.ds(base, PR_PER_W)], idx_v)
    pltpu.async_copy(h_hbm.at[idx_v], rows_v, sem).wait()
    pltpu.sync_copy(rows_v, out.at[pl.ds(base, PR_PER_W)])


BLK = 1000


def _layer_body(relu, x_ref, s0_ref, s1_ref, c0_ref, c1_ref, wr_ref, wn_ref,
                b_ref, o_ref):
    cnt = jnp.maximum(c0_ref[0][:, :1] + c1_ref[0][:, :1], 1.0)
    agg = (s0_ref[0] + s1_ref[0]) / cnt
    h = jnp.dot(x_ref[...], wr_ref[...], preferred_element_type=jnp.float32)
    h = h + jnp.dot(agg, wn_ref[...], preferred_element_type=jnp.float32)
    h = h + b_ref[...]
    if relu:
        h = jnp.maximum(h, 0.0)
    o_ref[...] = h


def _sage_layer_tc(x, sums, cnts, w_root, w_neigh, b, relu):
    n = x.shape[0]
    grid = (n // BLK,)
    return pl.pallas_call(
        functools.partial(_layer_body, relu),
        grid=grid,
        in_specs=[
            pl.BlockSpec((BLK, D), lambda i: (i, 0)),
            pl.BlockSpec((1, BLK, D), lambda i: (0, i, 0)),
            pl.BlockSpec((1, BLK, D), lambda i: (1, i, 0)),
            pl.BlockSpec((1, BLK, D), lambda i: (0, i, 0)),
            pl.BlockSpec((1, BLK, D), lambda i: (1, i, 0)),
            pl.BlockSpec((D, D), lambda i: (0, 0)),
            pl.BlockSpec((D, D), lambda i: (0, 0)),
            pl.BlockSpec((1, D), lambda i: (0, 0)),
        ],
        out_specs=pl.BlockSpec((BLK, D), lambda i: (i, 0)),
        out_shape=jax.ShapeDtypeStruct((n, D), jnp.float32),
    )(x, sums, sums, cnts, cnts, w_root, w_neigh, b)


def _pred_body(hs_ref, hd_ref, p1_ref, pb1_ref, p2_ref, pb2_ref, p3_ref,
               pb3_ref, o_ref):
    z = hs_ref[...] * hd_ref[...]
    z = jnp.maximum(
        jnp.dot(z, p1_ref[...], preferred_element_type=jnp.float32)
        + pb1_ref[...], 0.0)
    z = jnp.maximum(
        jnp.dot(z, p2_ref[...], preferred_element_type=jnp.float32)
        + pb2_ref[...], 0.0)
    o_ref[...] = (
        jnp.dot(z, p3_ref[...], preferred_element_type=jnp.float32)
        + pb3_ref[...])


def _predict_tc(hs, hd, p1, pb1, p2, pb2, p3, pb3):
    return pl.pallas_call(
        _pred_body,
        out_shape=jax.ShapeDtypeStruct((PP, 1), jnp.float32),
    )(hs, hd, p1, pb1, p2, pb2, p3, pb3)


def kernel(x, edge_index, W_root1, W_neigh1, b1, W_root2, W_neigh2, b2,
           P1, pb1, P2, pb2, P3, pb3):
    src = edge_index[0].astype(jnp.int32)
    dst = edge_index[1].astype(jnp.int32)
    pad = E_PAD - EE
    src_p = jnp.concatenate([src, jnp.zeros((pad,), jnp.int32)]).reshape(-1, CH)
    dst_p = jnp.concatenate([dst, jnp.full((pad,), NN, jnp.int32)]).reshape(-1, CH)
    zrows = jnp.zeros((RPT, D), jnp.float32)
    ones = jnp.ones((CH, D), jnp.float32)

    sums1, cnts = _seg_sum_cnt(x, src_p, dst_p, zrows, ones)
    h = _sage_layer_tc(x, sums1, cnts, W_root1, W_neigh1,
                       b1.reshape(1, D), relu=True)
    (sums2,) = _seg_sum(h, src_p, dst_p, zrows, ones)
    h2 = _sage_layer_tc(h, sums2, cnts, W_root2, W_neigh2,
                        b2.reshape(1, D), relu=False)

    pidx = jnp.concatenate([src[:PP], dst[:PP]])
    rows = _gather_rows(h2, pidx)
    return _predict_tc(rows[:PP], rows[PP:], P1, pb1.reshape(1, D),
                       P2, pb2.reshape(1, D), P3, pb3.reshape(1, 1))


# speedup vs baseline: 3.6625x; 1.0939x over previous
import functools

import jax
import jax.numpy as jnp
from jax import lax
from jax.experimental import pallas as pl
from jax.experimental.pallas import tpu as pltpu
from jax.experimental.pallas import tpu_sc as plsc

NN = 10000
EE = 320000
D = 128
PP = 1024
NC, NS = 2, 16
NW = NC * NS
CH = 128
NCHUNK = 80
IBLK = 8
E_PAD = NW * NCHUNK * CH
N_ACC = 10240
RPT = N_ACC // NS

_MESH = plsc.VectorSubcoreMesh(core_axis_name="c", subcore_axis_name="s")


NBLK = NCHUNK // IBLK


def _make_seg_sum(with_cnt):
    out_type = [jax.ShapeDtypeStruct((NC, N_ACC, D), jnp.float32)]
    if with_cnt:
        out_type.append(jax.ShapeDtypeStruct((NC, N_ACC, D), jnp.float32))

    @functools.partial(
        pl.kernel,
        out_type=out_type,
        mesh=_MESH,
        scratch_types=[
            pltpu.VMEM((IBLK, CH), jnp.int32),
            pltpu.VMEM((IBLK, CH), jnp.int32),
            pltpu.VMEM((IBLK, CH), jnp.int32),
            pltpu.VMEM((IBLK, CH), jnp.int32),
            pltpu.VMEM((CH, D), jnp.float32),
            pltpu.VMEM((CH, D), jnp.float32),
            pltpu.VMEM_SHARED((N_ACC, D), jnp.float32),
            pltpu.SemaphoreType.DMA,
            pltpu.SemaphoreType.DMA,
            pltpu.SemaphoreType.DMA,
            pltpu.SemaphoreType.DMA,
        ],
    )
    def seg(table_hbm, src_hbm, dst_hbm, zrows_hbm, ones_hbm,
            *out_and_scratch):
        if with_cnt:
            out_sum, out_cnt = out_and_scratch[:2]
            rest = out_and_scratch[2:]
        else:
            out_sum = out_and_scratch[0]
            rest = out_and_scratch[1:]
        (src_a, src_b, dst_a, dst_b, rows0, rows1, acc_s,
         sg0, sg1, ss0, ss1) = rest
        rows = (rows0, rows1)
        sg = (sg0, sg1)
        ss = (ss0, ss1)
        c = lax.axis_index("c")
        s = lax.axis_index("s")
        w = s * NC + c
        base = s * RPT
        ibase = w * NCHUNK

        def wait_g(b):
            pltpu.make_async_copy(table_hbm.at[pl.ds(0, CH)], rows[b],
                                  sg[b]).wait()

        def wait_s(b):
            pltpu.make_async_copy(rows[b], acc_s.at[pl.ds(0, CH)],
                                  ss[b]).wait()

        pltpu.sync_copy(zrows_hbm, acc_s.at[pl.ds(base, RPT)])
        plsc.subcore_barrier()

        pltpu.sync_copy(src_hbm.at[pl.ds(ibase, IBLK)], src_a)
        pltpu.sync_copy(dst_hbm.at[pl.ds(ibase, IBLK)], dst_a)
        pltpu.async_copy(table_hbm.at[src_a.at[0]], rows[0], sg[0])

        def a_block(ko, src_cur, dst_cur, src_nxt, dst_nxt):
            @pl.when(ko < NBLK - 1)
            def _():
                pltpu.sync_copy(
                    src_hbm.at[pl.ds(ibase + (ko + 1) * IBLK, IBLK)], src_nxt)

            for i in range(IBLK):
                b = i % 2
                bo = 1 - b
                wait_g(b)
                pltpu.async_copy(rows[b], acc_s.at[dst_cur.at[i]], ss[b],
                                 add=True)
                if i == 0:
                    @pl.when(ko > 0)
                    def _():
                        wait_s(bo)
                else:
                    wait_s(bo)
                if i == 0:
                    @pl.when(ko < NBLK - 1)
                    def _():
                        pltpu.sync_copy(
                            dst_hbm.at[pl.ds(ibase + (ko + 1) * IBLK, IBLK)],
                            dst_nxt)
                if i < IBLK - 1:
                    pltpu.async_copy(table_hbm.at[src_cur.at[i + 1]],
                                     rows[bo], sg[bo])
                else:
                    @pl.when(ko < NBLK - 1)
                    def _():
                        pltpu.async_copy(table_hbm.at[src_nxt.at[0]],
                                         rows[bo], sg[bo])

        def a_outer(ko, carry):
            @pl.when(ko % 2 == 0)
            def _():
                a_block(ko, src_a, dst_a, src_b, dst_b)

            @pl.when(ko % 2 == 1)
            def _():
                a_block(ko, src_b, dst_b, src_a, dst_a)

            return carry

        lax.fori_loop(0, NBLK, a_outer, 0)
        wait_s((IBLK - 1) % 2)
        plsc.subcore_barrier()
        pltpu.sync_copy(acc_s.at[pl.ds(base, RPT)],
                        out_sum.at[c, pl.ds(base, RPT)])

        if with_cnt:
            pltpu.sync_copy(ones_hbm, rows0)
            pltpu.sync_copy(ones_hbm, rows1)
            plsc.subcore_barrier()
            pltpu.sync_copy(zrows_hbm, acc_s.at[pl.ds(base, RPT)])
            plsc.subcore_barrier()
            pltpu.sync_copy(dst_hbm.at[pl.ds(ibase, IBLK)], dst_a)

            def b_block(ko, dst_cur, dst_nxt):
                for i in range(IBLK):
                    b = i % 2
                    if i >= 2:
                        wait_s(b)
                    else:
                        @pl.when(ko > 0)
                        def _():
                            wait_s(b)
                    pltpu.async_copy(rows[b], acc_s.at[dst_cur.at[i]], ss[b],
                                     add=True)
                    if i == 1:
                        @pl.when(ko < NBLK - 1)
                        def _():
                            pltpu.sync_copy(
                                dst_hbm.at[pl.ds(ibase + (ko + 1) * IBLK,
                                                 IBLK)], dst_nxt)

            def b_outer(ko, carry):
                @pl.when(ko % 2 == 0)
                def _():
                    b_block(ko, dst_a, dst_b)

                @pl.when(ko % 2 == 1)
                def _():
                    b_block(ko, dst_b, dst_a)

                return carry

            lax.fori_loop(0, NBLK, b_outer, 0)
            wait_s(0)
            wait_s(1)
            plsc.subcore_barrier()
            pltpu.sync_copy(acc_s.at[pl.ds(base, RPT)],
                            out_cnt.at[c, pl.ds(base, RPT)])

    return seg


_seg_sum_cnt = _make_seg_sum(True)
_seg_sum = _make_seg_sum(False)


PRED_ROWS = 2 * PP
PR_PER_W = PRED_ROWS // NW


@functools.partial(
    pl.kernel,
    out_type=jax.ShapeDtypeStruct((PRED_ROWS, D), jnp.float32),
    mesh=_MESH,
    scratch_types=[
        pltpu.VMEM((PR_PER_W,), jnp.int32),
        pltpu.VMEM((PR_PER_W, D), jnp.float32),
        pltpu.SemaphoreType.DMA,
    ],
)
def _gather_rows(h_hbm, idx_hbm, out, idx_v, rows_v, sem):
    c = lax.axis_index("c")
    s = lax.axis_index("s")
    base = (s * NC + c) * PR_PER_W
    pltpu.sync_copy(idx_hbm.at[pl.ds(base, PR_PER_W)], idx_v)
    pltpu.async_copy(h_hbm.at[idx_v], rows_v, sem).wait()
    pltpu.sync_copy(rows_v, out.at[pl.ds(base, PR_PER_W)])


BLK = 1000


def _layer_body(relu, x_ref, s0_ref, s1_ref, c0_ref, c1_ref, wr_ref, wn_ref,
                b_ref, o_ref):
    cnt = jnp.maximum(c0_ref[0][:, :1] + c1_ref[0][:, :1], 1.0)
    agg = (s0_ref[0] + s1_ref[0]) / cnt
    h = jnp.dot(x_ref[...], wr_ref[...], preferred_element_type=jnp.float32)
    h = h + jnp.dot(agg, wn_ref[...], preferred_element_type=jnp.float32)
    h = h + b_ref[...]
    if relu:
        h = jnp.maximum(h, 0.0)
    o_ref[...] = h


def _sage_layer_tc(x, sums, cnts, w_root, w_neigh, b, relu):
    n = x.shape[0]
    grid = (n // BLK,)
    return pl.pallas_call(
        functools.partial(_layer_body, relu),
        grid=grid,
        in_specs=[
            pl.BlockSpec((BLK, D), lambda i: (i, 0)),
            pl.BlockSpec((1, BLK, D), lambda i: (0, i, 0)),
            pl.BlockSpec((1, BLK, D), lambda i: (1, i, 0)),
            pl.BlockSpec((1, BLK, D), lambda i: (0, i, 0)),
            pl.BlockSpec((1, BLK, D), lambda i: (1, i, 0)),
            pl.BlockSpec((D, D), lambda i: (0, 0)),
            pl.BlockSpec((D, D), lambda i: (0, 0)),
            pl.BlockSpec((1, D), lambda i: (0, 0)),
        ],
        out_specs=pl.BlockSpec((BLK, D), lambda i: (i, 0)),
        out_shape=jax.ShapeDtypeStruct((n, D), jnp.float32),
    )(x, sums, sums, cnts, cnts, w_root, w_neigh, b)


def _pred_body(hs_ref, hd_ref, p1_ref, pb1_ref, p2_ref, pb2_ref, p3_ref,
               pb3_ref, o_ref):
    z = hs_ref[...] * hd_ref[...]
    z = jnp.maximum(
        jnp.dot(z, p1_ref[...], preferred_element_type=jnp.float32)
        + pb1_ref[...], 0.0)
    z = jnp.maximum(
        jnp.dot(z, p2_ref[...], preferred_element_type=jnp.float32)
        + pb2_ref[...], 0.0)
    o_ref[...] = (
        jnp.dot(z, p3_ref[...], preferred_element_type=jnp.float32)
        + pb3_ref[...])


def _predict_tc(hs, hd, p1, pb1, p2, pb2, p3, pb3):
    return pl.pallas_call(
        _pred_body,
        out_shape=jax.ShapeDtypeStruct((PP, 1), jnp.float32),
    )(hs, hd, p1, pb1, p2, pb2, p3, pb3)


def kernel(x, edge_index, W_root1, W_neigh1, b1, W_root2, W_neigh2, b2,
           P1, pb1, P2, pb2, P3, pb3):
    src = edge_index[0].astype(jnp.int32)
    dst = edge_index[1].astype(jnp.int32)
    pad = E_PAD - EE
    src_p = jnp.concatenate([src, jnp.zeros((pad,), jnp.int32)]).reshape(-1, CH)
    dst_p = jnp.concatenate([dst, jnp.full((pad,), NN, jnp.int32)]).reshape(-1, CH)
    zrows = jnp.zeros((RPT, D), jnp.float32)
    ones = jnp.ones((CH, D), jnp.float32)

    sums1, cnts = _seg_sum_cnt(x, src_p, dst_p, zrows, ones)
    h = _sage_layer_tc(x, sums1, cnts, W_root1, W_neigh1,
                       b1.reshape(1, D), relu=True)
    (sums2,) = _seg_sum(h, src_p, dst_p, zrows, ones)
    h2 = _sage_layer_tc(h, sums2, cnts, W_root2, W_neigh2,
                        b2.reshape(1, D), relu=False)

    pidx = jnp.concatenate([src[:PP], dst[:PP]])
    rows = _gather_rows(h2, pidx)
    return _predict_tc(rows[:PP], rows[PP:], P1, pb1.reshape(1, D),
                       P2, pb2.reshape(1, D), P3, pb3.reshape(1, 1))
